# Initial kernel scaffold; baseline (speedup 1.0000x reference)
import jax
import jax.numpy as jnp
from jax.experimental import pallas as pl


def kernel(x, embed, W1, b1, W2, b2):
    raise NotImplementedError("write your pallas kernel here")


# speedup vs baseline: 1.7496x; 1.7496x over previous
import functools

import jax
import jax.numpy as jnp
from jax import lax
from jax.experimental import pallas as pl
from jax.experimental.pallas import tpu as pltpu
from jax.experimental.pallas import tpu_sc as plsc

V, D, H, C = 1000000, 32, 100, 100
B, L = 4096, 200

NC, NS = 2, 16
NW = NC * NS
RPW = B // NW


def _make_cbow_sum():
    mesh = plsc.VectorSubcoreMesh(core_axis_name="c", subcore_axis_name="s")

    @functools.partial(
        pl.kernel,
        mesh=mesh,
        compiler_params=pltpu.CompilerParams(use_tc_tiling_on_sc=False),
        out_type=jax.ShapeDtypeStruct((B, D), jnp.float32),
        scratch_types=[
            pltpu.VMEM((L,), jnp.int32),
            pltpu.VMEM((L, D), jnp.float32),
            pltpu.VMEM((RPW, D), jnp.float32),
            pltpu.SemaphoreType.DMA,
        ],
    )
    def cbow_sum(embed_hbm, x_hbm, out_hbm, idx_row, buf_v, out_v, sem):
        wid = lax.axis_index("s") * NC + lax.axis_index("c")
        base = wid * RPW

        def row(r, _):
            pltpu.sync_copy(x_hbm.at[base + r], idx_row)
            pltpu.async_copy(embed_hbm.at[idx_row], buf_v, sem).wait()

            def acc(j, carry):
                a0, a1 = carry
                return a0 + buf_v[j, 0:16], a1 + buf_v[j, 16:32]

            a0, a1 = lax.fori_loop(
                0, L, acc,
                (jnp.zeros((16,), jnp.float32), jnp.zeros((16,), jnp.float32)),
            )
            out_v[r, 0:16] = a0
            out_v[r, 16:32] = a1
            return 0

        lax.fori_loop(0, RPW, row, 0)
        pltpu.sync_copy(out_v, out_hbm.at[pl.ds(base, RPW)])

    return cbow_sum


_cbow_sum = _make_cbow_sum()


def _mlp_body(s_ref, w1_ref, b1_ref, w2_ref, b2_ref, out_ref):
    h = jnp.dot(s_ref[...], w1_ref[...], preferred_element_type=jnp.float32)
    h = jnp.maximum(h + b1_ref[...], 0.0)
    out_ref[...] = (
        jnp.dot(h, w2_ref[...], preferred_element_type=jnp.float32) + b2_ref[...]
    )


def kernel(x, embed, W1, b1, W2, b2):
    s = _cbow_sum(embed, x)
    out = pl.pallas_call(
        _mlp_body,
        out_shape=jax.ShapeDtypeStruct((B, C), jnp.float32),
    )(s, W1, b1.reshape(1, H), W2, b2.reshape(1, C))
    return out


# speedup vs baseline: 2.2347x; 1.2772x over previous
import functools

import jax
import jax.numpy as jnp
from jax import lax
from jax.experimental import pallas as pl
from jax.experimental.pallas import tpu as pltpu
from jax.experimental.pallas import tpu_sc as plsc

V, D, H, C = 1000000, 32, 100, 100
B, L = 4096, 200

NC, NS = 2, 16
NW = NC * NS
RPW = B // NW


def _make_cbow_sum():
    mesh = plsc.VectorSubcoreMesh(core_axis_name="c", subcore_axis_name="s")

    @functools.partial(
        pl.kernel,
        mesh=mesh,
        compiler_params=pltpu.CompilerParams(use_tc_tiling_on_sc=False),
        out_type=jax.ShapeDtypeStruct((B, D), jnp.float32),
        scratch_types=[
            pltpu.VMEM((L,), jnp.int32),
            pltpu.VMEM((L,), jnp.int32),
            pltpu.VMEM((L, D), jnp.float32),
            pltpu.VMEM((L, D), jnp.float32),
            pltpu.VMEM((RPW, D), jnp.float32),
            pltpu.SemaphoreType.DMA,
            pltpu.SemaphoreType.DMA,
            pltpu.SemaphoreType.DMA,
            pltpu.SemaphoreType.DMA,
        ],
    )
    def cbow_sum(embed_hbm, x_hbm, out_hbm,
                 idx0, idx1, buf0, buf1, out_v, gsem0, gsem1, isem0, isem1):
        wid = lax.axis_index("s") * NC + lax.axis_index("c")
        base = wid * RPW
        idx = (idx0, idx1)
        buf = (buf0, buf1)
        gsem = (gsem0, gsem1)
        isem = (isem0, isem1)

        pltpu.sync_copy(x_hbm.at[base], idx0)
        pltpu.make_async_copy(embed_hbm.at[idx0], buf0, gsem0).start()
        pltpu.make_async_copy(x_hbm.at[base + 1], idx1, isem1).start()

        def sum_rows(b, r):
            def acc(j, carry):
                a = list(carry)
                for q in range(4):
                    a[q] = a[q] + b[4 * j + q, 0:16]
                    a[4 + q] = a[4 + q] + b[4 * j + q, 16:32]
                return tuple(a)

            z = tuple(jnp.zeros((16,), jnp.float32) for _ in range(8))
            a = lax.fori_loop(0, L // 4, acc, z, unroll=5)
            out_v[r, 0:16] = (a[0] + a[1]) + (a[2] + a[3])
            out_v[r, 16:32] = (a[4] + a[5]) + (a[6] + a[7])

        def step(r, p):
            rn = r + 1
            rf = r + 2

            @pl.when(rn < RPW)
            def _():
                pltpu.make_async_copy(x_hbm.at[base + rn], idx[1 - p],
                                      isem[1 - p]).wait()
                pltpu.make_async_copy(embed_hbm.at[idx[1 - p]], buf[1 - p],
                                      gsem[1 - p]).start()

            pltpu.make_async_copy(embed_hbm.at[idx[p]], buf[p], gsem[p]).wait()

            @pl.when(rf < RPW)
            def _():
                pltpu.make_async_copy(x_hbm.at[base + rf], idx[p],
                                      isem[p]).start()

            sum_rows(buf[p], r)

        def pair(r2, _):
            step(2 * r2, 0)
            step(2 * r2 + 1, 1)
            return 0

        lax.fori_loop(0, RPW // 2, pair, 0)
        pltpu.sync_copy(out_v, out_hbm.at[pl.ds(base, RPW)])

    return cbow_sum


_cbow_sum = _make_cbow_sum()


def _mlp_body(s_ref, w1_ref, b1_ref, w2_ref, b2_ref, out_ref):
    h = jnp.dot(s_ref[...], w1_ref[...], preferred_element_type=jnp.float32)
    h = jnp.maximum(h + b1_ref[...], 0.0)
    out_ref[...] = (
        jnp.dot(h, w2_ref[...], preferred_element_type=jnp.float32) + b2_ref[...]
    )


def kernel(x, embed, W1, b1, W2, b2):
    s = _cbow_sum(embed, x)
    out = pl.pallas_call(
        _mlp_body,
        out_shape=jax.ShapeDtypeStruct((B, C), jnp.float32),
    )(s, W1, b1.reshape(1, H), W2, b2.reshape(1, C))
    return out
